# Initial kernel scaffold; baseline (speedup 1.0000x reference)
import jax
import jax.numpy as jnp
from jax.experimental import pallas as pl


def kernel(title_ids, text_ids, title_table, text_table):
    raise NotImplementedError("write your pallas kernel here")


# speedup vs baseline: 11.3933x; 11.3933x over previous
import functools

import jax
import jax.numpy as jnp
from jax import lax
from jax.experimental import pallas as pl
from jax.experimental.pallas import tpu as pltpu
from jax.experimental.pallas import tpu_sc as plsc

NC = 2
NS = 16
NW = NC * NS
LANES = 16
GRP = 128
CHUNK = 64


def _splat_lane(vec, lane):
    idx = jnp.full((LANES, 1), lane, jnp.int32)
    dn = lax.GatherDimensionNumbers(
        offset_dims=(), collapsed_slice_dims=(0,), start_index_map=(0,))
    return lax.gather(vec, idx, dn, (1,),
                      mode=lax.GatherScatterMode.PROMISE_IN_BOUNDS)


def _seg_div(i_vec, seq):
    assert seq == 50
    return lax.shift_right_logical(i_vec * 1311, 16)


def _sc_body(seq, nch, v_pad, tids_hbm, ids_hbm, ttab_hbm, xtab_hbm,
             out_hbm, vidx, maskf, seg, g3, acc, cnt, tidx, tbuf, obuf,
             acc_sh, cnt_sh):
    ngrp = (CHUNK * seq) // GRP
    sid = lax.axis_index("s")
    wid = sid * NC + lax.axis_index("c")
    sbase = sid * CHUNK

    def seg_body(j, _):
        for k in range(GRP // LANES):
            i_vec = jnp.arange(LANES, dtype=jnp.int32) + (j * GRP + k * LANES)
            seg[j, pl.ds(k * LANES, LANES)] = _seg_div(i_vec, seq) + sbase
        return _
    lax.fori_loop(0, ngrp, seg_body, None)

    def chunk_body(c, _):
        g = wid * nch + c
        pltpu.sync_copy(ids_hbm.at[g], vidx)
        pltpu.sync_copy(tids_hbm.at[g], tidx)

        def mask_body(j, _):
            for k in range(GRP // LANES):
                sl = pl.ds(k * LANES, LANES)
                v = vidx[j, sl]
                m = v == 0
                vidx[j, sl] = jnp.where(m, v_pad, v)
                maskf[j, sl] = jnp.where(m, 0.0, 1.0).astype(jnp.float32)
            return _
        lax.fori_loop(0, ngrp, mask_body, None)

        zf = jnp.zeros((LANES,), jnp.float32)
        def zero_body(b, _):
            acc[b, pl.ds(0, LANES)] = zf
            acc[b, pl.ds(LANES, LANES)] = zf
            return _
        lax.fori_loop(0, CHUNK, zero_body, None)
        for k in range(CHUNK // LANES):
            cnt[pl.ds(k * LANES, LANES)] = zf
        pltpu.sync_copy(acc, acc_sh.at[pl.ds(sbase, CHUNK)])
        pltpu.sync_copy(cnt, cnt_sh.at[pl.ds(sbase, CHUNK)])

        def gs_body(j, _):
            pltpu.sync_copy(xtab_hbm.at[vidx.at[j]], g3.at[j])
            pltpu.sync_copy(g3.at[j], acc_sh.at[seg.at[j]], add=True)
            pltpu.sync_copy(maskf.at[j], cnt_sh.at[seg.at[j]], add=True)
            return _
        lax.fori_loop(0, ngrp, gs_body, None)

        pltpu.sync_copy(ttab_hbm.at[tidx], tbuf)

        pltpu.sync_copy(acc_sh.at[pl.ds(sbase, CHUNK)], acc)
        pltpu.sync_copy(cnt_sh.at[pl.ds(sbase, CHUNK)], cnt)

        for k in range(CHUNK // LANES):
            sl = pl.ds(k * LANES, LANES)
            cnt[sl] = 1.0 / jnp.maximum(cnt[sl], 1.0)

        for k in range(CHUNK // LANES):
            cv = cnt[pl.ds(k * LANES, LANES)]
            for b2 in range(LANES):
                b = k * LANES + b2
                iv = _splat_lane(cv, b2)
                o = b * 64
                obuf[pl.ds(o, LANES)] = tbuf[b, pl.ds(0, LANES)]
                obuf[pl.ds(o + 16, LANES)] = tbuf[b, pl.ds(LANES, LANES)]
                obuf[pl.ds(o + 32, LANES)] = acc[b, pl.ds(0, LANES)] * iv
                obuf[pl.ds(o + 48, LANES)] = acc[b, pl.ds(LANES, LANES)] * iv

        pltpu.sync_copy(obuf, out_hbm.at[g])
        return _
    lax.fori_loop(0, nch, chunk_body, None)


def kernel(title_ids, text_ids, title_table, text_table):
    b, seq = text_ids.shape
    d = title_table.shape[1]
    v = text_table.shape[0]
    assert d == 32 and b % NW == 0 and (CHUNK * seq) % GRP == 0
    nch = b // (NW * CHUNK)
    ngrp = (CHUNK * seq) // GRP
    nglobal = b // CHUNK

    xtab = jnp.concatenate(
        [text_table, jnp.zeros((1, d), text_table.dtype)], axis=0)
    ids3 = text_ids.astype(jnp.int32).reshape(nglobal, ngrp, GRP)
    tids2 = title_ids.astype(jnp.int32).reshape(nglobal, CHUNK)

    mesh = plsc.VectorSubcoreMesh(core_axis_name="c", subcore_axis_name="s")
    run = pl.kernel(
        functools.partial(_sc_body, seq, nch, v),
        out_type=jax.ShapeDtypeStruct((nglobal, CHUNK * 2 * d), jnp.float32),
        mesh=mesh,
        scratch_types=[
            pltpu.VMEM((ngrp, GRP), jnp.int32),
            pltpu.VMEM((ngrp, GRP), jnp.float32),
            pltpu.VMEM((ngrp, GRP), jnp.int32),
            pltpu.VMEM((ngrp, GRP, d), jnp.float32),
            pltpu.VMEM((CHUNK, d), jnp.float32),
            pltpu.VMEM((CHUNK,), jnp.float32),
            pltpu.VMEM((CHUNK,), jnp.int32),
            pltpu.VMEM((CHUNK, d), jnp.float32),
            pltpu.VMEM((CHUNK * 2 * d,), jnp.float32),
            pltpu.VMEM_SHARED((NS * CHUNK, d), jnp.float32),
            pltpu.VMEM_SHARED((NS * CHUNK,), jnp.float32),
        ],
        compiler_params=pltpu.CompilerParams(use_tc_tiling_on_sc=False),
    )
    out = run(tids2, ids3, title_table, xtab)
    return out.reshape(b, 2 * d)
